# baseline (speedup 1.0000x reference)
import functools

import jax
import jax.numpy as jnp
from jax import lax
from jax.experimental import pallas as pl
from jax.experimental.pallas import tpu as pltpu
from jax.experimental.pallas import tpu_sc as plsc

N_NODES = 10000
D = 128

NC = 2
NS = 16
NW = NC * NS

NP = 10240
ROWS_PER_TILE = NP // NS
CHUNK = 128
N_EDGES = 320000
EP = 327680
NCH = EP // (NW * CHUNK)

def _deg_body(dst_hbm, degp_hbm, deg_sh, idx_v, ones_v, zb):
    c = lax.axis_index("c")
    s = lax.axis_index("s")
    w = c * NS + s

    for k in range(ROWS_PER_TILE // 16):
        zb[pl.ds(k * 16, 16)] = jnp.zeros((16,), jnp.float32)
    for k in range(CHUNK // 16):
        ones_v[pl.ds(k * 16, 16)] = jnp.ones((16,), jnp.float32)
    pltpu.sync_copy(zb, deg_sh.at[pl.ds(s * ROWS_PER_TILE, ROWS_PER_TILE)])
    pltpu.sync_copy(dst_hbm.at[pl.ds(w * NCH, NCH)], idx_v)
    plsc.subcore_barrier()

    def body(j, carry):
        pltpu.sync_copy(ones_v, deg_sh.at[idx_v.at[j]], add=True)
        return carry

    lax.fori_loop(0, NCH, body, 0)
    plsc.subcore_barrier()
    pltpu.sync_copy(
        deg_sh.at[pl.ds(s * ROWS_PER_TILE, ROWS_PER_TILE)],
        degp_hbm.at[c, pl.ds(s * ROWS_PER_TILE, ROWS_PER_TILE)],
    )


ROUND = 40


def _agg_body(src_hbm, dst_hbm, y_hbm, aggp_hbm, agg_sh, sidx, didx, rows,
              sem0, sem1):
    c = lax.axis_index("c")
    s = lax.axis_index("s")

    def zrow(i, carry):
        for k in range(D // 16):
            rows[0, i, pl.ds(k * 16, 16)] = jnp.zeros((16,), jnp.float32)
        return carry

    with jax.named_scope("ph_zero"):
        @pl.when(c == 0)
        def _():
            for t in range(ROWS_PER_TILE // CHUNK):
                r = s * ROWS_PER_TILE + t * CHUNK
                pltpu.sync_copy(y_hbm.at[pl.ds(r, CHUNK)],
                                agg_sh.at[pl.ds(r, CHUNK)])

        @pl.when(c != 0)
        def _():
            lax.fori_loop(0, CHUNK, zrow, 0)
            for t in range(ROWS_PER_TILE // CHUNK):
                r = s * ROWS_PER_TILE + t * CHUNK
                pltpu.sync_copy(rows.at[0], agg_sh.at[pl.ds(r, CHUNK)])
        plsc.subcore_barrier()

    def step2(jj, carry):
        j0 = jj * 2
        pltpu.async_copy(y_hbm.at[sidx.at[j0 + 1]], rows.at[1], sem1)
        pltpu.make_async_copy(y_hbm.at[sidx.at[j0]], rows.at[0], sem0).wait()
        pltpu.sync_copy(rows.at[0], agg_sh.at[didx.at[j0]], add=True)
        @pl.when(j0 + 2 < ROUND)
        def _():
            pltpu.async_copy(y_hbm.at[sidx.at[j0 + 2]], rows.at[0], sem0)
        pltpu.make_async_copy(y_hbm.at[sidx.at[j0 + 1]], rows.at[1], sem1).wait()
        pltpu.sync_copy(rows.at[1], agg_sh.at[didx.at[j0 + 1]], add=True)
        return carry

    w = c * NS + s
    with jax.named_scope("ph_edges"):
        for r in range(NCH // ROUND):
            pltpu.sync_copy(src_hbm.at[pl.ds(w * NCH + r * ROUND, ROUND)], sidx)
            pltpu.sync_copy(dst_hbm.at[pl.ds(w * NCH + r * ROUND, ROUND)], didx)
            pltpu.async_copy(y_hbm.at[sidx.at[0]], rows.at[0], sem0)
            lax.fori_loop(0, ROUND // 2, step2, 0)
    with jax.named_scope("ph_barrier"):
        plsc.subcore_barrier()
    with jax.named_scope("ph_out"):
        for t in range(ROWS_PER_TILE // CHUNK):
            r = s * ROWS_PER_TILE + t * CHUNK
            pltpu.sync_copy(
                agg_sh.at[pl.ds(r, CHUNK)], aggp_hbm.at[c, pl.ds(r, CHUNK)]
            )


def _scale_body(x_ref, degp_ref, y_ref, dis_ref):
    i = pl.program_id(0)
    deg = (degp_ref[0, pl.ds(i * 1024, 1024)]
           + degp_ref[1, pl.ds(i * 1024, 1024)] + 1.0)
    dis = lax.rsqrt(deg)
    y_ref[...] = dis[:, None] * x_ref[...]
    dis_ref[...] = dis[:, None]


def _scale_call(x_p, degp):
    return pl.pallas_call(
        _scale_body,
        grid=(NP // 1024,),
        in_specs=[
            pl.BlockSpec((1024, D), lambda i: (i, 0)),
            pl.BlockSpec((NC, NP), lambda i: (0, 0)),
        ],
        out_specs=[
            pl.BlockSpec((1024, D), lambda i: (i, 0)),
            pl.BlockSpec((1024, 1), lambda i: (i, 0)),
        ],
        out_shape=(
            jax.ShapeDtypeStruct((NP, D), jnp.float32),
            jax.ShapeDtypeStruct((NP, 1), jnp.float32),
        ),
    )(x_p, degp)


def _head_body(aggp_ref, dis_ref, w1_ref, b1_ref, w2_ref, b2_ref,
               x3_ref, x1_ref, x2_ref):
    agg = dis_ref[...] * (aggp_ref[0] + aggp_ref[1])
    x1 = jnp.dot(agg, w1_ref[...], preferred_element_type=jnp.float32) + b1_ref[...]
    x2 = jnp.dot(agg, w2_ref[...], preferred_element_type=jnp.float32) + b2_ref[...]
    s = x1 + x2
    m = jnp.max(s, axis=1, keepdims=True)
    z = s - m
    x3_ref[...] = z - jnp.log(jnp.sum(jnp.exp(z), axis=1, keepdims=True))
    x1_ref[...] = x1
    x2_ref[...] = x2


def _head_call(aggp, dis, W1, b1, W2, b2):
    blk = 1000
    out = jax.ShapeDtypeStruct((N_NODES, D), jnp.float32)
    return pl.pallas_call(
        _head_body,
        grid=(N_NODES // blk,),
        in_specs=[
            pl.BlockSpec((NC, blk, D), lambda i: (0, i, 0)),
            pl.BlockSpec((blk, 1), lambda i: (i, 0)),
            pl.BlockSpec((D, D), lambda i: (0, 0)),
            pl.BlockSpec((1, D), lambda i: (0, 0)),
            pl.BlockSpec((D, D), lambda i: (0, 0)),
            pl.BlockSpec((1, D), lambda i: (0, 0)),
        ],
        out_specs=[
            pl.BlockSpec((blk, D), lambda i: (i, 0)),
            pl.BlockSpec((blk, D), lambda i: (i, 0)),
            pl.BlockSpec((blk, D), lambda i: (i, 0)),
        ],
        out_shape=(out, out, out),
    )(aggp, dis, W1, b1, W2, b2)


@functools.lru_cache(maxsize=1)
def _sc_kernels():
    mesh = plsc.VectorSubcoreMesh(
        core_axis_name="c", subcore_axis_name="s", num_cores=NC, num_subcores=NS
    )
    deg_kernel = pl.kernel(
        _deg_body,
        out_type=jax.ShapeDtypeStruct((NC, NP), jnp.float32),
        mesh=mesh,
        scratch_types=[
            pltpu.VMEM_SHARED((NP,), jnp.float32),
            pltpu.VMEM((NCH, CHUNK), jnp.int32),
            pltpu.VMEM((CHUNK,), jnp.float32),
            pltpu.VMEM((ROWS_PER_TILE,), jnp.float32),
        ],
    )
    agg_kernel = pl.kernel(
        _agg_body,
        out_type=jax.ShapeDtypeStruct((NC, NP, D), jnp.float32),
        mesh=mesh,
        scratch_types=[
            pltpu.VMEM_SHARED((NP, D), jnp.float32),
            pltpu.VMEM((ROUND, CHUNK), jnp.int32),
            pltpu.VMEM((ROUND, CHUNK), jnp.int32),
            pltpu.VMEM((2, CHUNK, D), jnp.float32),
            pltpu.SemaphoreType.DMA,
            pltpu.SemaphoreType.DMA,
        ],
    )
    return deg_kernel, agg_kernel


def kernel(x, edge_index, W1, b1, W2, b2):
    src = edge_index[0].astype(jnp.int32)
    dst = edge_index[1].astype(jnp.int32)
    pad = EP - N_EDGES
    ar = jnp.arange(pad, dtype=jnp.int32)
    src_p = jnp.concatenate([src, ar % N_NODES]).reshape(NW * NCH, CHUNK)
    dst_p = jnp.concatenate([dst, N_NODES + ar % (NP - N_NODES)]
                            ).reshape(NW * NCH, CHUNK)
    x_p = jnp.zeros((NP, D), jnp.float32).at[:N_NODES].set(x)

    deg_kernel, agg_kernel = _sc_kernels()
    dst_k1 = jnp.concatenate(
        [dst, N_NODES + (ar * 7) % (NP - N_NODES)]).reshape(NW * NCH, CHUNK)
    degp = deg_kernel(dst_k1)
    y, dis = _scale_call(x_p, degp)
    aggp = agg_kernel(src_p, dst_p, y)
    x3, x1, x2 = _head_call(aggp, dis, W1, b1.reshape(1, D),
                            W2, b2.reshape(1, D))
    return (x3, x1, x2)


# speedup vs baseline: 1.0132x; 1.0132x over previous
import functools

import jax
import jax.numpy as jnp
from jax import lax
from jax.experimental import pallas as pl
from jax.experimental.pallas import tpu as pltpu
from jax.experimental.pallas import tpu_sc as plsc

N_NODES = 10000
D = 128

NC = 2
NS = 16
NW = NC * NS

NP = 10240
ROWS_PER_TILE = NP // NS
CHUNK = 128
N_EDGES = 320000
EP = 327680
NCH = EP // (NW * CHUNK)

def _deg_body(dst_hbm, degp_hbm, deg_sh, idx_v, ones_v, zb):
    c = lax.axis_index("c")
    s = lax.axis_index("s")
    w = c * NS + s

    for k in range(ROWS_PER_TILE // 16):
        zb[pl.ds(k * 16, 16)] = jnp.zeros((16,), jnp.float32)
    for k in range(CHUNK // 16):
        ones_v[pl.ds(k * 16, 16)] = jnp.ones((16,), jnp.float32)
    pltpu.sync_copy(zb, deg_sh.at[pl.ds(s * ROWS_PER_TILE, ROWS_PER_TILE)])
    pltpu.sync_copy(dst_hbm.at[pl.ds(w * NCH, NCH)], idx_v)
    plsc.subcore_barrier()

    def body(j, carry):
        pltpu.sync_copy(ones_v, deg_sh.at[idx_v.at[j]], add=True)
        return carry

    lax.fori_loop(0, NCH, body, 0)
    plsc.subcore_barrier()
    pltpu.sync_copy(
        deg_sh.at[pl.ds(s * ROWS_PER_TILE, ROWS_PER_TILE)],
        degp_hbm.at[c, pl.ds(s * ROWS_PER_TILE, ROWS_PER_TILE)],
    )


ROUND = 40


def _agg_body(src_hbm, dst_hbm, y_hbm, aggp_hbm, agg_sh, sidx, didx, rows,
              sem0, sem1):
    c = lax.axis_index("c")
    s = lax.axis_index("s")

    def zrow(i, carry):
        for k in range(D // 16):
            rows[0, i, pl.ds(k * 16, 16)] = jnp.zeros((16,), jnp.float32)
        return carry

    with jax.named_scope("ph_zero"):
        lax.fori_loop(0, CHUNK, zrow, 0)
        for t in range(ROWS_PER_TILE // CHUNK):
            r = s * ROWS_PER_TILE + t * CHUNK
            pltpu.sync_copy(rows.at[0], agg_sh.at[pl.ds(r, CHUNK)])
        plsc.subcore_barrier()

    def step2(jj, carry):
        j0 = jj * 2
        pltpu.async_copy(y_hbm.at[sidx.at[j0 + 1]], rows.at[1], sem1)
        pltpu.make_async_copy(y_hbm.at[sidx.at[j0]], rows.at[0], sem0).wait()
        pltpu.sync_copy(rows.at[0], agg_sh.at[didx.at[j0]], add=True)
        @pl.when(j0 + 2 < ROUND)
        def _():
            pltpu.async_copy(y_hbm.at[sidx.at[j0 + 2]], rows.at[0], sem0)
        pltpu.make_async_copy(y_hbm.at[sidx.at[j0 + 1]], rows.at[1], sem1).wait()
        pltpu.sync_copy(rows.at[1], agg_sh.at[didx.at[j0 + 1]], add=True)
        return carry

    w = c * NS + s
    with jax.named_scope("ph_edges"):
        for r in range(NCH // ROUND):
            pltpu.sync_copy(src_hbm.at[pl.ds(w * NCH + r * ROUND, ROUND)], sidx)
            pltpu.sync_copy(dst_hbm.at[pl.ds(w * NCH + r * ROUND, ROUND)], didx)
            pltpu.async_copy(y_hbm.at[sidx.at[0]], rows.at[0], sem0)
            lax.fori_loop(0, ROUND // 2, step2, 0)
    with jax.named_scope("ph_barrier"):
        plsc.subcore_barrier()
    with jax.named_scope("ph_out"):
        for t in range(ROWS_PER_TILE // CHUNK):
            r = s * ROWS_PER_TILE + t * CHUNK
            pltpu.sync_copy(
                agg_sh.at[pl.ds(r, CHUNK)], aggp_hbm.at[c, pl.ds(r, CHUNK)]
            )


def _scale_body(x_ref, degp_ref, y_ref, dis_ref):
    i = pl.program_id(0)
    deg = (degp_ref[0, pl.ds(i * 1024, 1024)]
           + degp_ref[1, pl.ds(i * 1024, 1024)] + 1.0)
    dis = lax.rsqrt(deg)
    y_ref[...] = dis[:, None] * x_ref[...]
    dis_ref[...] = dis[:, None]


def _scale_call(x_p, degp):
    return pl.pallas_call(
        _scale_body,
        grid=(NP // 1024,),
        in_specs=[
            pl.BlockSpec((1024, D), lambda i: (i, 0)),
            pl.BlockSpec((NC, NP), lambda i: (0, 0)),
        ],
        out_specs=[
            pl.BlockSpec((1024, D), lambda i: (i, 0)),
            pl.BlockSpec((1024, 1), lambda i: (i, 0)),
        ],
        out_shape=(
            jax.ShapeDtypeStruct((NP, D), jnp.float32),
            jax.ShapeDtypeStruct((NP, 1), jnp.float32),
        ),
    )(x_p, degp)


def _head_body(aggp_ref, y_ref, dis_ref, w1_ref, b1_ref, w2_ref, b2_ref,
               x3_ref, x1_ref, x2_ref):
    agg = dis_ref[...] * (aggp_ref[0] + aggp_ref[1] + y_ref[...])
    x1 = jnp.dot(agg, w1_ref[...], preferred_element_type=jnp.float32) + b1_ref[...]
    x2 = jnp.dot(agg, w2_ref[...], preferred_element_type=jnp.float32) + b2_ref[...]
    s = x1 + x2
    m = jnp.max(s, axis=1, keepdims=True)
    z = s - m
    x3_ref[...] = z - jnp.log(jnp.sum(jnp.exp(z), axis=1, keepdims=True))
    x1_ref[...] = x1
    x2_ref[...] = x2


def _head_call(aggp, y, dis, W1, b1, W2, b2):
    blk = 1000
    out = jax.ShapeDtypeStruct((N_NODES, D), jnp.float32)
    return pl.pallas_call(
        _head_body,
        grid=(N_NODES // blk,),
        in_specs=[
            pl.BlockSpec((NC, blk, D), lambda i: (0, i, 0)),
            pl.BlockSpec((blk, D), lambda i: (i, 0)),
            pl.BlockSpec((blk, 1), lambda i: (i, 0)),
            pl.BlockSpec((D, D), lambda i: (0, 0)),
            pl.BlockSpec((1, D), lambda i: (0, 0)),
            pl.BlockSpec((D, D), lambda i: (0, 0)),
            pl.BlockSpec((1, D), lambda i: (0, 0)),
        ],
        out_specs=[
            pl.BlockSpec((blk, D), lambda i: (i, 0)),
            pl.BlockSpec((blk, D), lambda i: (i, 0)),
            pl.BlockSpec((blk, D), lambda i: (i, 0)),
        ],
        out_shape=(out, out, out),
    )(aggp, y, dis, W1, b1, W2, b2)


@functools.lru_cache(maxsize=1)
def _sc_kernels():
    mesh = plsc.VectorSubcoreMesh(
        core_axis_name="c", subcore_axis_name="s", num_cores=NC, num_subcores=NS
    )
    deg_kernel = pl.kernel(
        _deg_body,
        out_type=jax.ShapeDtypeStruct((NC, NP), jnp.float32),
        mesh=mesh,
        scratch_types=[
            pltpu.VMEM_SHARED((NP,), jnp.float32),
            pltpu.VMEM((NCH, CHUNK), jnp.int32),
            pltpu.VMEM((CHUNK,), jnp.float32),
            pltpu.VMEM((ROWS_PER_TILE,), jnp.float32),
        ],
    )
    agg_kernel = pl.kernel(
        _agg_body,
        out_type=jax.ShapeDtypeStruct((NC, NP, D), jnp.float32),
        mesh=mesh,
        scratch_types=[
            pltpu.VMEM_SHARED((NP, D), jnp.float32),
            pltpu.VMEM((ROUND, CHUNK), jnp.int32),
            pltpu.VMEM((ROUND, CHUNK), jnp.int32),
            pltpu.VMEM((2, CHUNK, D), jnp.float32),
            pltpu.SemaphoreType.DMA,
            pltpu.SemaphoreType.DMA,
        ],
    )
    return deg_kernel, agg_kernel


def kernel(x, edge_index, W1, b1, W2, b2):
    src = edge_index[0].astype(jnp.int32)
    dst = edge_index[1].astype(jnp.int32)
    pad = EP - N_EDGES
    ar = jnp.arange(pad, dtype=jnp.int32)
    dst_p = jnp.concatenate([dst, N_NODES + ar % (NP - N_NODES)]
                            ).reshape(NW * NCH, CHUNK)

    deg_kernel, agg_kernel = _sc_kernels()
    degp = deg_kernel(dst_p)

    src_b, x_b = lax.optimization_barrier((src, x))
    src_p = jnp.concatenate([src_b, ar % N_NODES]).reshape(NW * NCH, CHUNK)
    x_p = jnp.zeros((NP, D), jnp.float32).at[:N_NODES].set(x_b)

    y, dis = _scale_call(x_p, degp)
    aggp = agg_kernel(src_p, dst_p, y)
    x3, x1, x2 = _head_call(aggp, y, dis, W1, b1.reshape(1, D),
                            W2, b2.reshape(1, D))
    return (x3, x1, x2)


# speedup vs baseline: 1.0154x; 1.0022x over previous
import functools

import jax
import jax.numpy as jnp
from jax import lax
from jax.experimental import pallas as pl
from jax.experimental.pallas import tpu as pltpu
from jax.experimental.pallas import tpu_sc as plsc

N_NODES = 10000
D = 128

NC = 2
NS = 16
NW = NC * NS

NP = 10240
ROWS_PER_TILE = NP // NS
CHUNK = 128
N_EDGES = 320000
EP = 327680
NCH = EP // (NW * CHUNK)

def _deg_body(dst_hbm, degp_hbm, deg_sh, idx_v, ones_v, zb):
    c = lax.axis_index("c")
    s = lax.axis_index("s")
    w = c * NS + s

    for k in range(ROWS_PER_TILE // 16):
        zb[pl.ds(k * 16, 16)] = jnp.zeros((16,), jnp.float32)
    for k in range(CHUNK // 16):
        ones_v[pl.ds(k * 16, 16)] = jnp.ones((16,), jnp.float32)
    pltpu.sync_copy(zb, deg_sh.at[pl.ds(s * ROWS_PER_TILE, ROWS_PER_TILE)])
    pltpu.sync_copy(dst_hbm.at[pl.ds(w * NCH, NCH)], idx_v)
    plsc.subcore_barrier()

    def body(j, carry):
        pltpu.sync_copy(ones_v, deg_sh.at[idx_v.at[j]], add=True)
        return carry

    lax.fori_loop(0, NCH, body, 0)
    plsc.subcore_barrier()
    pltpu.sync_copy(
        deg_sh.at[pl.ds(s * ROWS_PER_TILE, ROWS_PER_TILE)],
        degp_hbm.at[c, pl.ds(s * ROWS_PER_TILE, ROWS_PER_TILE)],
    )


ROUND = 40


def _agg_body(src_hbm, dst_hbm, y_hbm, aggp_hbm, agg_sh, sidx, didx, rows,
              sem0, sem1):
    c = lax.axis_index("c")
    s = lax.axis_index("s")

    def zrow(i, carry):
        for k in range(D // 16):
            rows[0, i, pl.ds(k * 16, 16)] = jnp.zeros((16,), jnp.float32)
        return carry

    with jax.named_scope("ph_zero"):
        lax.fori_loop(0, CHUNK, zrow, 0)
        for t in range(ROWS_PER_TILE // CHUNK):
            r = s * ROWS_PER_TILE + t * CHUNK
            pltpu.sync_copy(rows.at[0], agg_sh.at[pl.ds(r, CHUNK)])
        plsc.subcore_barrier()

    def step2(jj, carry):
        j0 = jj * 2
        pltpu.async_copy(
            y_hbm.at[sidx.at[pl.ds((j0 + 1) * CHUNK, CHUNK)]], rows.at[1], sem1)
        pltpu.make_async_copy(
            y_hbm.at[sidx.at[pl.ds(j0 * CHUNK, CHUNK)]], rows.at[0], sem0).wait()
        pltpu.sync_copy(rows.at[0], agg_sh.at[didx.at[j0]], add=True)
        @pl.when(j0 + 2 < ROUND)
        def _():
            pltpu.async_copy(
                y_hbm.at[sidx.at[pl.ds((j0 + 2) * CHUNK, CHUNK)]],
                rows.at[0], sem0)
        pltpu.make_async_copy(
            y_hbm.at[sidx.at[pl.ds((j0 + 1) * CHUNK, CHUNK)]],
            rows.at[1], sem1).wait()
        pltpu.sync_copy(rows.at[1], agg_sh.at[didx.at[j0 + 1]], add=True)
        return carry

    w = c * NS + s
    with jax.named_scope("ph_edges"):
        for r in range(NCH // ROUND):
            pltpu.sync_copy(
                src_hbm.at[pl.ds((w * NCH + r * ROUND) * CHUNK, ROUND * CHUNK)],
                sidx)
            pltpu.sync_copy(dst_hbm.at[pl.ds(w * NCH + r * ROUND, ROUND)], didx)
            pltpu.async_copy(y_hbm.at[sidx.at[pl.ds(0, CHUNK)]], rows.at[0], sem0)
            lax.fori_loop(0, ROUND // 2, step2, 0)
    with jax.named_scope("ph_barrier"):
        plsc.subcore_barrier()
    with jax.named_scope("ph_out"):
        for t in range(ROWS_PER_TILE // CHUNK):
            r = s * ROWS_PER_TILE + t * CHUNK
            pltpu.sync_copy(
                agg_sh.at[pl.ds(r, CHUNK)], aggp_hbm.at[c, pl.ds(r, CHUNK)]
            )


def _scale_body(x_ref, degp_ref, y_ref, dis_ref):
    i = pl.program_id(0)
    deg = (degp_ref[0, pl.ds(i * 1024, 1024)]
           + degp_ref[1, pl.ds(i * 1024, 1024)] + 1.0)
    dis = lax.rsqrt(deg)
    y_ref[...] = dis[:, None] * x_ref[...]
    dis_ref[...] = dis[:, None]


def _scale_call(x_p, degp):
    return pl.pallas_call(
        _scale_body,
        grid=(NP // 1024,),
        in_specs=[
            pl.BlockSpec((1024, D), lambda i: (i, 0)),
            pl.BlockSpec((NC, NP), lambda i: (0, 0)),
        ],
        out_specs=[
            pl.BlockSpec((1024, D), lambda i: (i, 0)),
            pl.BlockSpec((1024, 1), lambda i: (i, 0)),
        ],
        out_shape=(
            jax.ShapeDtypeStruct((NP, D), jnp.float32),
            jax.ShapeDtypeStruct((NP, 1), jnp.float32),
        ),
    )(x_p, degp)


def _head_body(aggp_ref, y_ref, dis_ref, w1_ref, b1_ref, w2_ref, b2_ref,
               x3_ref, x1_ref, x2_ref):
    agg = dis_ref[...] * (aggp_ref[0] + aggp_ref[1] + y_ref[...])
    x1 = jnp.dot(agg, w1_ref[...], preferred_element_type=jnp.float32) + b1_ref[...]
    x2 = jnp.dot(agg, w2_ref[...], preferred_element_type=jnp.float32) + b2_ref[...]
    s = x1 + x2
    m = jnp.max(s, axis=1, keepdims=True)
    z = s - m
    x3_ref[...] = z - jnp.log(jnp.sum(jnp.exp(z), axis=1, keepdims=True))
    x1_ref[...] = x1
    x2_ref[...] = x2


def _head_call(aggp, y, dis, W1, b1, W2, b2):
    blk = 1000
    out = jax.ShapeDtypeStruct((N_NODES, D), jnp.float32)
    return pl.pallas_call(
        _head_body,
        grid=(N_NODES // blk,),
        in_specs=[
            pl.BlockSpec((NC, blk, D), lambda i: (0, i, 0)),
            pl.BlockSpec((blk, D), lambda i: (i, 0)),
            pl.BlockSpec((blk, 1), lambda i: (i, 0)),
            pl.BlockSpec((D, D), lambda i: (0, 0)),
            pl.BlockSpec((1, D), lambda i: (0, 0)),
            pl.BlockSpec((D, D), lambda i: (0, 0)),
            pl.BlockSpec((1, D), lambda i: (0, 0)),
        ],
        out_specs=[
            pl.BlockSpec((blk, D), lambda i: (i, 0)),
            pl.BlockSpec((blk, D), lambda i: (i, 0)),
            pl.BlockSpec((blk, D), lambda i: (i, 0)),
        ],
        out_shape=(out, out, out),
    )(aggp, y, dis, W1, b1, W2, b2)


@functools.lru_cache(maxsize=1)
def _sc_kernels():
    mesh = plsc.VectorSubcoreMesh(
        core_axis_name="c", subcore_axis_name="s", num_cores=NC, num_subcores=NS
    )
    deg_kernel = pl.kernel(
        _deg_body,
        out_type=jax.ShapeDtypeStruct((NC, NP), jnp.float32),
        mesh=mesh,
        scratch_types=[
            pltpu.VMEM_SHARED((NP,), jnp.float32),
            pltpu.VMEM((NCH, CHUNK), jnp.int32),
            pltpu.VMEM((CHUNK,), jnp.float32),
            pltpu.VMEM((ROWS_PER_TILE,), jnp.float32),
        ],
    )
    agg_kernel = pl.kernel(
        _agg_body,
        out_type=jax.ShapeDtypeStruct((NC, NP, D), jnp.float32),
        mesh=mesh,
        scratch_types=[
            pltpu.VMEM_SHARED((NP, D), jnp.float32),
            pltpu.VMEM((ROUND * CHUNK,), jnp.int32),
            pltpu.VMEM((ROUND, CHUNK), jnp.int32),
            pltpu.VMEM((2, CHUNK, D), jnp.float32),
            pltpu.SemaphoreType.DMA,
            pltpu.SemaphoreType.DMA,
        ],
    )
    return deg_kernel, agg_kernel


def kernel(x, edge_index, W1, b1, W2, b2):
    src = edge_index[0].astype(jnp.int32)
    dst = edge_index[1].astype(jnp.int32)
    pad = EP - N_EDGES
    ar = jnp.arange(pad, dtype=jnp.int32)
    dst_p = jnp.concatenate([dst, N_NODES + ar % (NP - N_NODES)]
                            ).reshape(NW * NCH, CHUNK)

    deg_kernel, agg_kernel = _sc_kernels()
    degp = deg_kernel(dst_p)

    src_b, x_b = lax.optimization_barrier((src, x))
    src_p = jnp.concatenate([src_b, ar % N_NODES])
    x_p = jnp.zeros((NP, D), jnp.float32).at[:N_NODES].set(x_b)

    y, dis = _scale_call(x_p, degp)
    aggp = agg_kernel(src_p, dst_p, y)
    x3, x1, x2 = _head_call(aggp, y, dis, W1, b1.reshape(1, D),
                            W2, b2.reshape(1, D))
    return (x3, x1, x2)


# speedup vs baseline: 1.0499x; 1.0340x over previous
import functools

import jax
import jax.numpy as jnp
from jax import lax
from jax.experimental import pallas as pl
from jax.experimental.pallas import tpu as pltpu
from jax.experimental.pallas import tpu_sc as plsc

N_NODES = 10000
D = 128

NC = 2
NS = 16
NW = NC * NS

NP = 10240
ROWS_PER_TILE = NP // NS
CHUNK = 128
N_EDGES = 320000
EP = 327680
NCH = EP // (NW * CHUNK)

def _deg_body(dst_hbm, degp_hbm, deg_sh, idx_v, ones_v, zb, dsem0, dsem1):
    c = lax.axis_index("c")
    s = lax.axis_index("s")
    w = c * NS + s

    for k in range(ROWS_PER_TILE // 16):
        zb[pl.ds(k * 16, 16)] = jnp.zeros((16,), jnp.float32)
    for k in range(CHUNK // 16):
        ones_v[pl.ds(k * 16, 16)] = jnp.ones((16,), jnp.float32)
    pltpu.sync_copy(zb, deg_sh.at[pl.ds(s * ROWS_PER_TILE, ROWS_PER_TILE)])
    pltpu.sync_copy(dst_hbm.at[pl.ds(w * NCH, NCH)], idx_v)
    plsc.subcore_barrier()

    def body(jj, carry):
        j0 = jj * 2
        pltpu.async_copy(ones_v, deg_sh.at[idx_v.at[j0]], dsem0, add=True)
        pltpu.async_copy(ones_v, deg_sh.at[idx_v.at[j0 + 1]], dsem1, add=True)
        pltpu.make_async_copy(ones_v, deg_sh.at[idx_v.at[j0]], dsem0).wait()
        pltpu.make_async_copy(ones_v, deg_sh.at[idx_v.at[j0 + 1]], dsem1).wait()
        return carry

    lax.fori_loop(0, NCH // 2, body, 0)
    plsc.subcore_barrier()
    pltpu.sync_copy(
        deg_sh.at[pl.ds(s * ROWS_PER_TILE, ROWS_PER_TILE)],
        degp_hbm.at[c, pl.ds(s * ROWS_PER_TILE, ROWS_PER_TILE)],
    )


ROUND = 40


def _agg_body(src_hbm, dst_hbm, y_hbm, aggp_hbm, agg_sh, sidx, didx, rows,
              sem0, sem1):
    c = lax.axis_index("c")
    s = lax.axis_index("s")

    def zrow(i, carry):
        for k in range(D // 16):
            rows[0, i, pl.ds(k * 16, 16)] = jnp.zeros((16,), jnp.float32)
        return carry

    with jax.named_scope("ph_zero"):
        lax.fori_loop(0, CHUNK, zrow, 0)
        for t in range(ROWS_PER_TILE // CHUNK):
            r = s * ROWS_PER_TILE + t * CHUNK
            pltpu.sync_copy(rows.at[0], agg_sh.at[pl.ds(r, CHUNK)])
        plsc.subcore_barrier()

    def step2(jj, carry):
        j0 = jj * 2
        pltpu.async_copy(
            y_hbm.at[sidx.at[pl.ds((j0 + 1) * CHUNK, CHUNK)]], rows.at[1], sem1)
        pltpu.make_async_copy(
            y_hbm.at[sidx.at[pl.ds(j0 * CHUNK, CHUNK)]], rows.at[0], sem0).wait()
        pltpu.sync_copy(rows.at[0], agg_sh.at[didx.at[j0]], add=True)
        @pl.when(j0 + 2 < ROUND)
        def _():
            pltpu.async_copy(
                y_hbm.at[sidx.at[pl.ds((j0 + 2) * CHUNK, CHUNK)]],
                rows.at[0], sem0)
        pltpu.make_async_copy(
            y_hbm.at[sidx.at[pl.ds((j0 + 1) * CHUNK, CHUNK)]],
            rows.at[1], sem1).wait()
        pltpu.sync_copy(rows.at[1], agg_sh.at[didx.at[j0 + 1]], add=True)
        return carry

    w = c * NS + s
    with jax.named_scope("ph_edges"):
        for r in range(NCH // ROUND):
            pltpu.sync_copy(
                src_hbm.at[pl.ds((w * NCH + r * ROUND) * CHUNK, ROUND * CHUNK)],
                sidx)
            pltpu.sync_copy(dst_hbm.at[pl.ds(w * NCH + r * ROUND, ROUND)], didx)
            pltpu.async_copy(y_hbm.at[sidx.at[pl.ds(0, CHUNK)]], rows.at[0], sem0)
            lax.fori_loop(0, ROUND // 2, step2, 0)
    with jax.named_scope("ph_barrier"):
        plsc.subcore_barrier()
    with jax.named_scope("ph_out"):
        for t in range(ROWS_PER_TILE // CHUNK):
            r = s * ROWS_PER_TILE + t * CHUNK
            pltpu.sync_copy(
                agg_sh.at[pl.ds(r, CHUNK)], aggp_hbm.at[c, pl.ds(r, CHUNK)]
            )


def _scale_body(x_ref, degp_ref, y_ref, dis_ref):
    i = pl.program_id(0)
    deg = (degp_ref[0, pl.ds(i * 1024, 1024)]
           + degp_ref[1, pl.ds(i * 1024, 1024)] + 1.0)
    dis = lax.rsqrt(deg)
    y_ref[...] = dis[:, None] * x_ref[...]
    dis_ref[...] = dis[:, None]


def _scale_call(x_p, degp):
    return pl.pallas_call(
        _scale_body,
        grid=(NP // 1024,),
        in_specs=[
            pl.BlockSpec((1024, D), lambda i: (i, 0)),
            pl.BlockSpec((NC, NP), lambda i: (0, 0)),
        ],
        out_specs=[
            pl.BlockSpec((1024, D), lambda i: (i, 0)),
            pl.BlockSpec((1024, 1), lambda i: (i, 0)),
        ],
        out_shape=(
            jax.ShapeDtypeStruct((NP, D), jnp.float32),
            jax.ShapeDtypeStruct((NP, 1), jnp.float32),
        ),
    )(x_p, degp)


def _head_body(aggp_ref, y_ref, dis_ref, w1_ref, b1_ref, w2_ref, b2_ref,
               x3_ref, x1_ref, x2_ref):
    agg = dis_ref[...] * (aggp_ref[0] + aggp_ref[1] + y_ref[...])
    x1 = jnp.dot(agg, w1_ref[...], preferred_element_type=jnp.float32) + b1_ref[...]
    x2 = jnp.dot(agg, w2_ref[...], preferred_element_type=jnp.float32) + b2_ref[...]
    s = x1 + x2
    m = jnp.max(s, axis=1, keepdims=True)
    z = s - m
    x3_ref[...] = z - jnp.log(jnp.sum(jnp.exp(z), axis=1, keepdims=True))
    x1_ref[...] = x1
    x2_ref[...] = x2


def _head_call(aggp, y, dis, W1, b1, W2, b2):
    blk = 2000
    out = jax.ShapeDtypeStruct((N_NODES, D), jnp.float32)
    return pl.pallas_call(
        _head_body,
        grid=(N_NODES // blk,),
        in_specs=[
            pl.BlockSpec((NC, blk, D), lambda i: (0, i, 0)),
            pl.BlockSpec((blk, D), lambda i: (i, 0)),
            pl.BlockSpec((blk, 1), lambda i: (i, 0)),
            pl.BlockSpec((D, D), lambda i: (0, 0)),
            pl.BlockSpec((1, D), lambda i: (0, 0)),
            pl.BlockSpec((D, D), lambda i: (0, 0)),
            pl.BlockSpec((1, D), lambda i: (0, 0)),
        ],
        out_specs=[
            pl.BlockSpec((blk, D), lambda i: (i, 0)),
            pl.BlockSpec((blk, D), lambda i: (i, 0)),
            pl.BlockSpec((blk, D), lambda i: (i, 0)),
        ],
        out_shape=(out, out, out),
    )(aggp, y, dis, W1, b1, W2, b2)


@functools.lru_cache(maxsize=1)
def _sc_kernels():
    mesh = plsc.VectorSubcoreMesh(
        core_axis_name="c", subcore_axis_name="s", num_cores=NC, num_subcores=NS
    )
    deg_kernel = pl.kernel(
        _deg_body,
        out_type=jax.ShapeDtypeStruct((NC, NP), jnp.float32),
        mesh=mesh,
        scratch_types=[
            pltpu.VMEM_SHARED((NP,), jnp.float32),
            pltpu.VMEM((NCH, CHUNK), jnp.int32),
            pltpu.VMEM((CHUNK,), jnp.float32),
            pltpu.VMEM((ROWS_PER_TILE,), jnp.float32),
            pltpu.SemaphoreType.DMA,
            pltpu.SemaphoreType.DMA,
        ],
    )
    agg_kernel = pl.kernel(
        _agg_body,
        out_type=jax.ShapeDtypeStruct((NC, NP, D), jnp.float32),
        mesh=mesh,
        scratch_types=[
            pltpu.VMEM_SHARED((NP, D), jnp.float32),
            pltpu.VMEM((ROUND * CHUNK,), jnp.int32),
            pltpu.VMEM((ROUND, CHUNK), jnp.int32),
            pltpu.VMEM((2, CHUNK, D), jnp.float32),
            pltpu.SemaphoreType.DMA,
            pltpu.SemaphoreType.DMA,
        ],
    )
    return deg_kernel, agg_kernel


def kernel(x, edge_index, W1, b1, W2, b2):
    src = edge_index[0].astype(jnp.int32)
    dst = edge_index[1].astype(jnp.int32)
    pad = EP - N_EDGES
    ar = jnp.arange(pad, dtype=jnp.int32)
    dst_p = jnp.concatenate([dst, N_NODES + ar % (NP - N_NODES)]
                            ).reshape(NW * NCH, CHUNK)

    deg_kernel, agg_kernel = _sc_kernels()
    degp = deg_kernel(dst_p)

    src_b, x_b = lax.optimization_barrier((src, x))
    src_p = jnp.concatenate([src_b, ar % N_NODES])
    x_p = jnp.zeros((NP, D), jnp.float32).at[:N_NODES].set(x_b)

    y, dis = _scale_call(x_p, degp)
    aggp = agg_kernel(src_p, dst_p, y)
    x3, x1, x2 = _head_call(aggp, y, dis, W1, b1.reshape(1, D),
                            W2, b2.reshape(1, D))
    return (x3, x1, x2)


# speedup vs baseline: 1.0675x; 1.0168x over previous
import functools

import jax
import jax.numpy as jnp
from jax import lax
from jax.experimental import pallas as pl
from jax.experimental.pallas import tpu as pltpu
from jax.experimental.pallas import tpu_sc as plsc

N_NODES = 10000
D = 128

NC = 2
NS = 16
NW = NC * NS

NP = 10240
ROWS_PER_TILE = NP // NS
CHUNK = 128
N_EDGES = 320000
EP = 327680
NCH = EP // (NW * CHUNK)

def _deg_body(dst_hbm, degp_hbm, deg_sh, idx_v, ones_v, zb, dsem0, dsem1):
    c = lax.axis_index("c")
    s = lax.axis_index("s")
    w = c * NS + s

    for k in range(ROWS_PER_TILE // 16):
        zb[pl.ds(k * 16, 16)] = jnp.zeros((16,), jnp.float32)
    for k in range(CHUNK // 16):
        ones_v[pl.ds(k * 16, 16)] = jnp.ones((16,), jnp.float32)
    pltpu.sync_copy(zb, deg_sh.at[pl.ds(s * ROWS_PER_TILE, ROWS_PER_TILE)])
    pltpu.sync_copy(dst_hbm.at[pl.ds(w * NCH, NCH)], idx_v)
    plsc.subcore_barrier()

    def body(jj, carry):
        j0 = jj * 4
        for q in range(4):
            sem = dsem0 if q % 2 == 0 else dsem1
            pltpu.async_copy(ones_v, deg_sh.at[idx_v.at[j0 + q]], sem, add=True)
        for q in range(4):
            sem = dsem0 if q % 2 == 0 else dsem1
            pltpu.make_async_copy(ones_v, deg_sh.at[idx_v.at[j0 + q]], sem).wait()
        return carry

    lax.fori_loop(0, NCH // 4, body, 0)
    plsc.subcore_barrier()
    pltpu.sync_copy(
        deg_sh.at[pl.ds(s * ROWS_PER_TILE, ROWS_PER_TILE)],
        degp_hbm.at[c, pl.ds(s * ROWS_PER_TILE, ROWS_PER_TILE)],
    )


ROUND = 40


def _agg_body(src_hbm, dst_hbm, y_hbm, aggp_hbm, agg_sh, sidx, didx, rows,
              sem0, sem1):
    c = lax.axis_index("c")
    s = lax.axis_index("s")

    def zrow(i, carry):
        for k in range(D // 16):
            rows[0, i, pl.ds(k * 16, 16)] = jnp.zeros((16,), jnp.float32)
        return carry

    with jax.named_scope("ph_zero"):
        lax.fori_loop(0, CHUNK, zrow, 0)
        for t in range(ROWS_PER_TILE // CHUNK):
            r = s * ROWS_PER_TILE + t * CHUNK
            pltpu.sync_copy(rows.at[0], agg_sh.at[pl.ds(r, CHUNK)])
        plsc.subcore_barrier()

    def step2(jj, carry):
        j0 = jj * 2
        pltpu.async_copy(
            y_hbm.at[sidx.at[pl.ds((j0 + 1) * CHUNK, CHUNK)]], rows.at[1], sem1)
        pltpu.make_async_copy(
            y_hbm.at[sidx.at[pl.ds(j0 * CHUNK, CHUNK)]], rows.at[0], sem0).wait()
        pltpu.sync_copy(rows.at[0], agg_sh.at[didx.at[j0]], add=True)
        @pl.when(j0 + 2 < ROUND)
        def _():
            pltpu.async_copy(
                y_hbm.at[sidx.at[pl.ds((j0 + 2) * CHUNK, CHUNK)]],
                rows.at[0], sem0)
        pltpu.make_async_copy(
            y_hbm.at[sidx.at[pl.ds((j0 + 1) * CHUNK, CHUNK)]],
            rows.at[1], sem1).wait()
        pltpu.sync_copy(rows.at[1], agg_sh.at[didx.at[j0 + 1]], add=True)
        return carry

    w = c * NS + s
    with jax.named_scope("ph_edges"):
        for r in range(NCH // ROUND):
            pltpu.sync_copy(
                src_hbm.at[pl.ds((w * NCH + r * ROUND) * CHUNK, ROUND * CHUNK)],
                sidx)
            pltpu.sync_copy(dst_hbm.at[pl.ds(w * NCH + r * ROUND, ROUND)], didx)
            pltpu.async_copy(y_hbm.at[sidx.at[pl.ds(0, CHUNK)]], rows.at[0], sem0)
            lax.fori_loop(0, ROUND // 2, step2, 0)
    with jax.named_scope("ph_barrier"):
        plsc.subcore_barrier()
    with jax.named_scope("ph_out"):
        for t in range(ROWS_PER_TILE // CHUNK):
            r = s * ROWS_PER_TILE + t * CHUNK
            pltpu.sync_copy(
                agg_sh.at[pl.ds(r, CHUNK)], aggp_hbm.at[c, pl.ds(r, CHUNK)]
            )


def _scale_body(x_ref, degp_ref, y_ref, dis_ref):
    i = pl.program_id(0)
    deg = (degp_ref[0, pl.ds(i * 2048, 2048)]
           + degp_ref[1, pl.ds(i * 2048, 2048)] + 1.0)
    dis = lax.rsqrt(deg)
    y_ref[...] = dis[:, None] * x_ref[...]
    dis_ref[...] = dis[:, None]


def _scale_call(x_p, degp):
    return pl.pallas_call(
        _scale_body,
        grid=(NP // 2048,),
        in_specs=[
            pl.BlockSpec((2048, D), lambda i: (i, 0)),
            pl.BlockSpec((NC, NP), lambda i: (0, 0)),
        ],
        out_specs=[
            pl.BlockSpec((2048, D), lambda i: (i, 0)),
            pl.BlockSpec((2048, 1), lambda i: (i, 0)),
        ],
        out_shape=(
            jax.ShapeDtypeStruct((NP, D), jnp.float32),
            jax.ShapeDtypeStruct((NP, 1), jnp.float32),
        ),
    )(x_p, degp)


def _head_body(aggp_ref, y_ref, dis_ref, w1_ref, b1_ref, w2_ref, b2_ref,
               x3_ref, x1_ref, x2_ref):
    agg = dis_ref[...] * (aggp_ref[0] + aggp_ref[1] + y_ref[...])
    x1 = jnp.dot(agg, w1_ref[...], preferred_element_type=jnp.float32) + b1_ref[...]
    x2 = jnp.dot(agg, w2_ref[...], preferred_element_type=jnp.float32) + b2_ref[...]
    s = x1 + x2
    m = jnp.max(s, axis=1, keepdims=True)
    z = s - m
    x3_ref[...] = z - jnp.log(jnp.sum(jnp.exp(z), axis=1, keepdims=True))
    x1_ref[...] = x1
    x2_ref[...] = x2


def _head_call(aggp, y, dis, W1, b1, W2, b2):
    blk = 2000
    out = jax.ShapeDtypeStruct((N_NODES, D), jnp.float32)
    return pl.pallas_call(
        _head_body,
        grid=(N_NODES // blk,),
        in_specs=[
            pl.BlockSpec((NC, blk, D), lambda i: (0, i, 0)),
            pl.BlockSpec((blk, D), lambda i: (i, 0)),
            pl.BlockSpec((blk, 1), lambda i: (i, 0)),
            pl.BlockSpec((D, D), lambda i: (0, 0)),
            pl.BlockSpec((1, D), lambda i: (0, 0)),
            pl.BlockSpec((D, D), lambda i: (0, 0)),
            pl.BlockSpec((1, D), lambda i: (0, 0)),
        ],
        out_specs=[
            pl.BlockSpec((blk, D), lambda i: (i, 0)),
            pl.BlockSpec((blk, D), lambda i: (i, 0)),
            pl.BlockSpec((blk, D), lambda i: (i, 0)),
        ],
        out_shape=(out, out, out),
    )(aggp, y, dis, W1, b1, W2, b2)


@functools.lru_cache(maxsize=1)
def _sc_kernels():
    mesh = plsc.VectorSubcoreMesh(
        core_axis_name="c", subcore_axis_name="s", num_cores=NC, num_subcores=NS
    )
    deg_kernel = pl.kernel(
        _deg_body,
        out_type=jax.ShapeDtypeStruct((NC, NP), jnp.float32),
        mesh=mesh,
        scratch_types=[
            pltpu.VMEM_SHARED((NP,), jnp.float32),
            pltpu.VMEM((NCH, CHUNK), jnp.int32),
            pltpu.VMEM((CHUNK,), jnp.float32),
            pltpu.VMEM((ROWS_PER_TILE,), jnp.float32),
            pltpu.SemaphoreType.DMA,
            pltpu.SemaphoreType.DMA,
        ],
    )
    agg_kernel = pl.kernel(
        _agg_body,
        out_type=jax.ShapeDtypeStruct((NC, NP, D), jnp.float32),
        mesh=mesh,
        scratch_types=[
            pltpu.VMEM_SHARED((NP, D), jnp.float32),
            pltpu.VMEM((ROUND * CHUNK,), jnp.int32),
            pltpu.VMEM((ROUND, CHUNK), jnp.int32),
            pltpu.VMEM((2, CHUNK, D), jnp.float32),
            pltpu.SemaphoreType.DMA,
            pltpu.SemaphoreType.DMA,
        ],
    )
    return deg_kernel, agg_kernel


def kernel(x, edge_index, W1, b1, W2, b2):
    src = edge_index[0].astype(jnp.int32)
    dst = edge_index[1].astype(jnp.int32)
    pad = EP - N_EDGES
    ar = jnp.arange(pad, dtype=jnp.int32)
    dst_p = jnp.concatenate([dst, N_NODES + ar % (NP - N_NODES)]
                            ).reshape(NW * NCH, CHUNK)

    deg_kernel, agg_kernel = _sc_kernels()
    degp = deg_kernel(dst_p)

    src_b, x_b = lax.optimization_barrier((src, x))
    src_p = jnp.concatenate([src_b, ar % N_NODES])
    x_p = jnp.zeros((NP, D), jnp.float32).at[:N_NODES].set(x_b)

    y, dis = _scale_call(x_p, degp)
    aggp = agg_kernel(src_p, dst_p, y)
    x3, x1, x2 = _head_call(aggp, y, dis, W1, b1.reshape(1, D),
                            W2, b2.reshape(1, D))
    return (x3, x1, x2)


# speedup vs baseline: 1.0720x; 1.0042x over previous
import functools

import jax
import jax.numpy as jnp
from jax import lax
from jax.experimental import pallas as pl
from jax.experimental.pallas import tpu as pltpu
from jax.experimental.pallas import tpu_sc as plsc

N_NODES = 10000
D = 128

NC = 2
NS = 16
NW = NC * NS

NP = 10240
ROWS_PER_TILE = NP // NS
CHUNK = 128
N_EDGES = 320000
EP = 327680
NCH = EP // (NW * CHUNK)

def _deg_body(dst_hbm, degp_hbm, deg_sh, idx_v, ones_v, zb, dsem0, dsem1):
    c = lax.axis_index("c")
    s = lax.axis_index("s")
    w = c * NS + s

    for k in range(ROWS_PER_TILE // 16):
        zb[pl.ds(k * 16, 16)] = jnp.zeros((16,), jnp.float32)
    for k in range(CHUNK // 16):
        ones_v[pl.ds(k * 16, 16)] = jnp.ones((16,), jnp.float32)
    pltpu.sync_copy(zb, deg_sh.at[pl.ds(s * ROWS_PER_TILE, ROWS_PER_TILE)])
    pltpu.sync_copy(dst_hbm.at[pl.ds(w * NCH, NCH)], idx_v)
    plsc.subcore_barrier()

    def body(jj, carry):
        j0 = jj * 8
        for q in range(8):
            sem = dsem0 if q % 2 == 0 else dsem1
            pltpu.async_copy(ones_v, deg_sh.at[idx_v.at[j0 + q]], sem, add=True)
        for q in range(8):
            sem = dsem0 if q % 2 == 0 else dsem1
            pltpu.make_async_copy(ones_v, deg_sh.at[idx_v.at[j0 + q]], sem).wait()
        return carry

    lax.fori_loop(0, NCH // 8, body, 0)
    plsc.subcore_barrier()
    pltpu.sync_copy(
        deg_sh.at[pl.ds(s * ROWS_PER_TILE, ROWS_PER_TILE)],
        degp_hbm.at[c, pl.ds(s * ROWS_PER_TILE, ROWS_PER_TILE)],
    )


ROUND = 40


def _agg_body(src_hbm, dst_hbm, y_hbm, aggp_hbm, agg_sh, sidx, didx, rows,
              sem0, sem1):
    c = lax.axis_index("c")
    s = lax.axis_index("s")

    def zrow(i, carry):
        for k in range(D // 16):
            rows[0, i, pl.ds(k * 16, 16)] = jnp.zeros((16,), jnp.float32)
        return carry

    with jax.named_scope("ph_zero"):
        lax.fori_loop(0, CHUNK, zrow, 0)
        for t in range(ROWS_PER_TILE // CHUNK):
            r = s * ROWS_PER_TILE + t * CHUNK
            pltpu.sync_copy(rows.at[0], agg_sh.at[pl.ds(r, CHUNK)])
        plsc.subcore_barrier()

    def step2(jj, carry):
        j0 = jj * 2
        pltpu.async_copy(
            y_hbm.at[sidx.at[pl.ds((j0 + 1) * CHUNK, CHUNK)]], rows.at[1], sem1)
        pltpu.make_async_copy(
            y_hbm.at[sidx.at[pl.ds(j0 * CHUNK, CHUNK)]], rows.at[0], sem0).wait()
        pltpu.sync_copy(rows.at[0], agg_sh.at[didx.at[j0]], add=True)
        @pl.when(j0 + 2 < ROUND)
        def _():
            pltpu.async_copy(
                y_hbm.at[sidx.at[pl.ds((j0 + 2) * CHUNK, CHUNK)]],
                rows.at[0], sem0)
        pltpu.make_async_copy(
            y_hbm.at[sidx.at[pl.ds((j0 + 1) * CHUNK, CHUNK)]],
            rows.at[1], sem1).wait()
        pltpu.sync_copy(rows.at[1], agg_sh.at[didx.at[j0 + 1]], add=True)
        return carry

    w = c * NS + s
    with jax.named_scope("ph_edges"):
        for r in range(NCH // ROUND):
            pltpu.sync_copy(
                src_hbm.at[pl.ds((w * NCH + r * ROUND) * CHUNK, ROUND * CHUNK)],
                sidx)
            pltpu.sync_copy(dst_hbm.at[pl.ds(w * NCH + r * ROUND, ROUND)], didx)
            pltpu.async_copy(y_hbm.at[sidx.at[pl.ds(0, CHUNK)]], rows.at[0], sem0)
            lax.fori_loop(0, ROUND // 2, step2, 0)
    with jax.named_scope("ph_barrier"):
        plsc.subcore_barrier()
    with jax.named_scope("ph_out"):
        for t in range(ROWS_PER_TILE // CHUNK):
            r = s * ROWS_PER_TILE + t * CHUNK
            pltpu.sync_copy(
                agg_sh.at[pl.ds(r, CHUNK)], aggp_hbm.at[c, pl.ds(r, CHUNK)]
            )


def _scale_body(x_ref, degp_ref, y_ref, dis_ref):
    i = pl.program_id(0)
    deg = (degp_ref[0, pl.ds(i * 2048, 2048)]
           + degp_ref[1, pl.ds(i * 2048, 2048)] + 1.0)
    dis = lax.rsqrt(deg)
    y_ref[...] = dis[:, None] * x_ref[...]
    dis_ref[...] = dis[:, None]


def _scale_call(x_p, degp):
    return pl.pallas_call(
        _scale_body,
        grid=(NP // 2048,),
        in_specs=[
            pl.BlockSpec((2048, D), lambda i: (i, 0)),
            pl.BlockSpec((NC, NP), lambda i: (0, 0)),
        ],
        out_specs=[
            pl.BlockSpec((2048, D), lambda i: (i, 0)),
            pl.BlockSpec((2048, 1), lambda i: (i, 0)),
        ],
        out_shape=(
            jax.ShapeDtypeStruct((NP, D), jnp.float32),
            jax.ShapeDtypeStruct((NP, 1), jnp.float32),
        ),
    )(x_p, degp)


def _head_body(aggp_ref, y_ref, dis_ref, w1_ref, b1_ref, w2_ref, b2_ref,
               x3_ref, x1_ref, x2_ref):
    agg = dis_ref[...] * (aggp_ref[0] + aggp_ref[1] + y_ref[...])
    x1 = jnp.dot(agg, w1_ref[...], preferred_element_type=jnp.float32) + b1_ref[...]
    x2 = jnp.dot(agg, w2_ref[...], preferred_element_type=jnp.float32) + b2_ref[...]
    s = x1 + x2
    m = jnp.max(s, axis=1, keepdims=True)
    z = s - m
    x3_ref[...] = z - jnp.log(jnp.sum(jnp.exp(z), axis=1, keepdims=True))
    x1_ref[...] = x1
    x2_ref[...] = x2


def _head_call(aggp, y, dis, W1, b1, W2, b2):
    blk = 2000
    out = jax.ShapeDtypeStruct((N_NODES, D), jnp.float32)
    return pl.pallas_call(
        _head_body,
        grid=(N_NODES // blk,),
        in_specs=[
            pl.BlockSpec((NC, blk, D), lambda i: (0, i, 0)),
            pl.BlockSpec((blk, D), lambda i: (i, 0)),
            pl.BlockSpec((blk, 1), lambda i: (i, 0)),
            pl.BlockSpec((D, D), lambda i: (0, 0)),
            pl.BlockSpec((1, D), lambda i: (0, 0)),
            pl.BlockSpec((D, D), lambda i: (0, 0)),
            pl.BlockSpec((1, D), lambda i: (0, 0)),
        ],
        out_specs=[
            pl.BlockSpec((blk, D), lambda i: (i, 0)),
            pl.BlockSpec((blk, D), lambda i: (i, 0)),
            pl.BlockSpec((blk, D), lambda i: (i, 0)),
        ],
        out_shape=(out, out, out),
    )(aggp, y, dis, W1, b1, W2, b2)


@functools.lru_cache(maxsize=1)
def _sc_kernels():
    mesh = plsc.VectorSubcoreMesh(
        core_axis_name="c", subcore_axis_name="s", num_cores=NC, num_subcores=NS
    )
    deg_kernel = pl.kernel(
        _deg_body,
        out_type=jax.ShapeDtypeStruct((NC, NP), jnp.float32),
        mesh=mesh,
        scratch_types=[
            pltpu.VMEM_SHARED((NP,), jnp.float32),
            pltpu.VMEM((NCH, CHUNK), jnp.int32),
            pltpu.VMEM((CHUNK,), jnp.float32),
            pltpu.VMEM((ROWS_PER_TILE,), jnp.float32),
            pltpu.SemaphoreType.DMA,
            pltpu.SemaphoreType.DMA,
        ],
    )
    agg_kernel = pl.kernel(
        _agg_body,
        out_type=jax.ShapeDtypeStruct((NC, NP, D), jnp.float32),
        mesh=mesh,
        scratch_types=[
            pltpu.VMEM_SHARED((NP, D), jnp.float32),
            pltpu.VMEM((ROUND * CHUNK,), jnp.int32),
            pltpu.VMEM((ROUND, CHUNK), jnp.int32),
            pltpu.VMEM((2, CHUNK, D), jnp.float32),
            pltpu.SemaphoreType.DMA,
            pltpu.SemaphoreType.DMA,
        ],
    )
    return deg_kernel, agg_kernel


def kernel(x, edge_index, W1, b1, W2, b2):
    src = edge_index[0].astype(jnp.int32)
    dst = edge_index[1].astype(jnp.int32)
    pad = EP - N_EDGES
    ar = jnp.arange(pad, dtype=jnp.int32)
    dst_p = jnp.concatenate([dst, N_NODES + ar % (NP - N_NODES)]
                            ).reshape(NW * NCH, CHUNK)

    deg_kernel, agg_kernel = _sc_kernels()
    degp = deg_kernel(dst_p)

    src_b, x_b = lax.optimization_barrier((src, x))
    src_p = jnp.concatenate([src_b, ar % N_NODES])
    x_p = jnp.zeros((NP, D), jnp.float32).at[:N_NODES].set(x_b)

    y, dis = _scale_call(x_p, degp)
    aggp = agg_kernel(src_p, dst_p, y)
    x3, x1, x2 = _head_call(aggp, y, dis, W1, b1.reshape(1, D),
                            W2, b2.reshape(1, D))
    return (x3, x1, x2)
